# initial kernel scaffold (unmeasured)
import jax
import jax.numpy as jnp
from jax import lax
from jax.experimental import pallas as pl
from jax.experimental.pallas import tpu as pltpu

N_DEV = 8
B_LOC = 2
SQ = 256
SKV = 256
H_LOC = 4
DH = 64
DM = 512
HD = H_LOC * DH
BLK = 64


def kernel(x, Wq, K_ext, V_ext, Wo):
    k2 = K_ext.reshape(16, SKV, 32 * DH)
    v2 = V_ext.reshape(16, SKV, 32 * DH)

    def body(x_ref, wq_ref, k_hbm, v_hbm, wo_ref, out_ref,
             wq_bufs, wo_bufs, k_chunk, v_chunk,
             ksem, vsem, ssem_q, rsem_q, ssem_o, rsem_o):
        my = lax.axis_index("i")
        right = lax.rem(my + 1, N_DEV)
        left = lax.rem(my + N_DEV - 1, N_DEV)

        def start_kv_fetch(r):
            origin = lax.rem(my + N_DEV - r, N_DEV)
            col = origin * HD
            p = r % 2
            kcp = pltpu.make_async_copy(
                k_hbm.at[pl.ds(my * B_LOC, B_LOC), :, pl.ds(col, HD)],
                k_chunk.at[p], ksem.at[p])
            vcp = pltpu.make_async_copy(
                v_hbm.at[pl.ds(my * B_LOC, B_LOC), :, pl.ds(col, HD)],
                v_chunk.at[p], vsem.at[p])
            kcp.start()
            vcp.start()

        def wait_kv(r):
            p = r % 2
            pltpu.make_async_copy(k_hbm.at[0, :, :HD], k_chunk.at[p],
                                  ksem.at[p]).wait()
            pltpu.make_async_copy(v_hbm.at[0, :, :HD], v_chunk.at[p],
                                  vsem.at[p]).wait()

        start_kv_fetch(0)

        barrier_sem = pltpu.get_barrier_semaphore()
        for nbr in (left, right):
            pl.semaphore_signal(barrier_sem, inc=1, device_id=(nbr,),
                                device_id_type=pl.DeviceIdType.MESH)
        pl.semaphore_wait(barrier_sem, 2)

        wq_bufs[0] = wq_ref[...]
        wo_bufs[0] = wo_ref[...]

        qb = lax.broadcasted_iota(jnp.int32, (SQ, SKV), 0) // BLK
        kb = lax.broadcasted_iota(jnp.int32, (SQ, SKV), 1) // BLK
        mask = (qb == kb) | (kb == 0) | ((qb + kb) % 3 == 0)

        x2 = x_ref[...].reshape(B_LOC * SQ, DM)

        def compute_chunk(r, acc):
            p = r % 2
            wq_c = wq_bufs[r]
            wo_c = wo_bufs[r]
            q_all = jnp.dot(x2, wq_c, preferred_element_type=jnp.float32)
            ctx_rows = []
            for b in range(B_LOC):
                parts = []
                for hh in range(H_LOC):
                    q_h = q_all[b * SQ:(b + 1) * SQ, hh * DH:(hh + 1) * DH]
                    k_h = k_chunk[p, b, :, hh * DH:(hh + 1) * DH]
                    v_h = v_chunk[p, b, :, hh * DH:(hh + 1) * DH]
                    s = lax.dot_general(
                        q_h, k_h, (((1,), (1,)), ((), ())),
                        preferred_element_type=jnp.float32) * 0.125
                    s = jnp.where(mask, s, -1e9)
                    m = jnp.max(s, axis=-1, keepdims=True)
                    w = jnp.exp(s - m)
                    w = w / jnp.sum(w, axis=-1, keepdims=True)
                    parts.append(jnp.dot(w, v_h,
                                         preferred_element_type=jnp.float32))
                ctx_rows.append(jnp.concatenate(parts, axis=1))
            ctx2 = jnp.concatenate(ctx_rows, axis=0)
            return acc + jnp.dot(ctx2, wo_c,
                                 preferred_element_type=jnp.float32)

        acc = jnp.zeros((B_LOC * SQ, DM), jnp.float32)

        for h in range(N_DEV - 1):
            rq = pltpu.make_async_remote_copy(
                src_ref=wq_bufs.at[h], dst_ref=wq_bufs.at[h + 1],
                send_sem=ssem_q.at[h], recv_sem=rsem_q.at[h + 1],
                device_id=(right,), device_id_type=pl.DeviceIdType.MESH)
            ro = pltpu.make_async_remote_copy(
                src_ref=wo_bufs.at[h], dst_ref=wo_bufs.at[h + 1],
                send_sem=ssem_o.at[h], recv_sem=rsem_o.at[h + 1],
                device_id=(right,), device_id_type=pl.DeviceIdType.MESH)
            rq.start()
            ro.start()
            start_kv_fetch(h + 1)
            wait_kv(h)
            acc = compute_chunk(h, acc)
            rq.wait()
            ro.wait()

        wait_kv(N_DEV - 1)
        acc = compute_chunk(N_DEV - 1, acc)
        out_ref[...] = acc.reshape(B_LOC, SQ, DM)

    return pl.pallas_call(
        body,
        out_shape=jax.ShapeDtypeStruct((B_LOC, SQ, DM), jnp.float32),
        in_specs=[
            pl.BlockSpec(memory_space=pltpu.VMEM),
            pl.BlockSpec(memory_space=pltpu.VMEM),
            pl.BlockSpec(memory_space=pltpu.ANY),
            pl.BlockSpec(memory_space=pltpu.ANY),
            pl.BlockSpec(memory_space=pltpu.VMEM),
        ],
        out_specs=pl.BlockSpec(memory_space=pltpu.VMEM),
        scratch_shapes=[
            pltpu.VMEM((N_DEV, DM, HD), jnp.float32),
            pltpu.VMEM((N_DEV, HD, DM), jnp.float32),
            pltpu.VMEM((2, B_LOC, SKV, HD), jnp.float32),
            pltpu.VMEM((2, B_LOC, SKV, HD), jnp.float32),
            pltpu.SemaphoreType.DMA((2,)),
            pltpu.SemaphoreType.DMA((2,)),
            pltpu.SemaphoreType.DMA((N_DEV,)),
            pltpu.SemaphoreType.DMA((N_DEV,)),
            pltpu.SemaphoreType.DMA((N_DEV,)),
            pltpu.SemaphoreType.DMA((N_DEV,)),
        ],
        compiler_params=pltpu.CompilerParams(collective_id=0),
    )(x, Wq, k2, v2, Wo)


# baseline (device time: 166875 ns/iter reference)
import jax
import jax.numpy as jnp
from jax import lax
from jax.experimental import pallas as pl
from jax.experimental.pallas import tpu as pltpu

N_DEV = 8
B_LOC = 2
SQ = 256
SKV = 256
H_LOC = 4
DH = 64
DM = 512
HD = H_LOC * DH
BLK = 64


def kernel(x, Wq, K_ext, V_ext, Wo):
    k2 = K_ext.reshape(16, SKV, 32 * DH)
    v2 = V_ext.reshape(16, SKV, 32 * DH)

    def body(x_ref, wq_ref, k_hbm, v_hbm, wo_ref, out_ref,
             wq_bufs, wo_bufs, k_chunk, v_chunk,
             ksem, vsem, ssem_q, rsem_q, ssem_o, rsem_o):
        my = lax.axis_index("i")
        right = lax.rem(my + 1, N_DEV)
        left = lax.rem(my + N_DEV - 1, N_DEV)

        def start_kv_fetch(r):
            origin = lax.rem(my + N_DEV - r, N_DEV)
            col = origin * HD
            p = r % 2
            kcp = pltpu.make_async_copy(
                k_hbm.at[pl.ds(my * B_LOC, B_LOC), :, pl.ds(col, HD)],
                k_chunk.at[p], ksem.at[p])
            vcp = pltpu.make_async_copy(
                v_hbm.at[pl.ds(my * B_LOC, B_LOC), :, pl.ds(col, HD)],
                v_chunk.at[p], vsem.at[p])
            kcp.start()
            vcp.start()

        def wait_kv(r):
            p = r % 2
            pltpu.make_async_copy(
                k_hbm.at[pl.ds(0, B_LOC), :, pl.ds(0, HD)],
                k_chunk.at[p], ksem.at[p]).wait()
            pltpu.make_async_copy(
                v_hbm.at[pl.ds(0, B_LOC), :, pl.ds(0, HD)],
                v_chunk.at[p], vsem.at[p]).wait()

        start_kv_fetch(0)

        barrier_sem = pltpu.get_barrier_semaphore()
        for nbr in (left, right):
            pl.semaphore_signal(barrier_sem, inc=1, device_id=(nbr,),
                                device_id_type=pl.DeviceIdType.MESH)
        pl.semaphore_wait(barrier_sem, 2)

        wq_bufs[0] = wq_ref[...]
        wo_bufs[0] = wo_ref[...]

        qb = lax.broadcasted_iota(jnp.int32, (SQ, SKV), 0) // BLK
        kb = lax.broadcasted_iota(jnp.int32, (SQ, SKV), 1) // BLK
        mask = (qb == kb) | (kb == 0) | ((qb + kb) % 3 == 0)

        x2 = x_ref[...].reshape(B_LOC * SQ, DM)

        def compute_chunk(r, acc):
            p = r % 2
            wq_c = wq_bufs[r]
            wo_c = wo_bufs[r]
            q_all = jnp.dot(x2, wq_c, preferred_element_type=jnp.float32)
            ctx_rows = []
            for b in range(B_LOC):
                parts = []
                for hh in range(H_LOC):
                    q_h = q_all[b * SQ:(b + 1) * SQ, hh * DH:(hh + 1) * DH]
                    k_h = k_chunk[p, b, :, hh * DH:(hh + 1) * DH]
                    v_h = v_chunk[p, b, :, hh * DH:(hh + 1) * DH]
                    s = lax.dot_general(
                        q_h, k_h, (((1,), (1,)), ((), ())),
                        preferred_element_type=jnp.float32) * 0.125
                    s = jnp.where(mask, s, -1e9)
                    m = jnp.max(s, axis=-1, keepdims=True)
                    w = jnp.exp(s - m)
                    w = w / jnp.sum(w, axis=-1, keepdims=True)
                    parts.append(jnp.dot(w, v_h,
                                         preferred_element_type=jnp.float32))
                ctx_rows.append(jnp.concatenate(parts, axis=1))
            ctx2 = jnp.concatenate(ctx_rows, axis=0)
            return acc + jnp.dot(ctx2, wo_c,
                                 preferred_element_type=jnp.float32)

        acc = jnp.zeros((B_LOC * SQ, DM), jnp.float32)

        for h in range(N_DEV - 1):
            rq = pltpu.make_async_remote_copy(
                src_ref=wq_bufs.at[h], dst_ref=wq_bufs.at[h + 1],
                send_sem=ssem_q.at[h], recv_sem=rsem_q.at[h + 1],
                device_id=(right,), device_id_type=pl.DeviceIdType.MESH)
            ro = pltpu.make_async_remote_copy(
                src_ref=wo_bufs.at[h], dst_ref=wo_bufs.at[h + 1],
                send_sem=ssem_o.at[h], recv_sem=rsem_o.at[h + 1],
                device_id=(right,), device_id_type=pl.DeviceIdType.MESH)
            rq.start()
            ro.start()
            start_kv_fetch(h + 1)
            wait_kv(h)
            acc = compute_chunk(h, acc)
            rq.wait()
            ro.wait()

        wait_kv(N_DEV - 1)
        acc = compute_chunk(N_DEV - 1, acc)
        out_ref[...] = acc.reshape(B_LOC, SQ, DM)

    return pl.pallas_call(
        body,
        out_shape=jax.ShapeDtypeStruct((B_LOC, SQ, DM), jnp.float32),
        in_specs=[
            pl.BlockSpec(memory_space=pltpu.VMEM),
            pl.BlockSpec(memory_space=pltpu.VMEM),
            pl.BlockSpec(memory_space=pl.ANY),
            pl.BlockSpec(memory_space=pl.ANY),
            pl.BlockSpec(memory_space=pltpu.VMEM),
        ],
        out_specs=pl.BlockSpec(memory_space=pltpu.VMEM),
        scratch_shapes=[
            pltpu.VMEM((N_DEV, DM, HD), jnp.float32),
            pltpu.VMEM((N_DEV, HD, DM), jnp.float32),
            pltpu.VMEM((2, B_LOC, SKV, HD), jnp.float32),
            pltpu.VMEM((2, B_LOC, SKV, HD), jnp.float32),
            pltpu.SemaphoreType.DMA((2,)),
            pltpu.SemaphoreType.DMA((2,)),
            pltpu.SemaphoreType.DMA((N_DEV,)),
            pltpu.SemaphoreType.DMA((N_DEV,)),
            pltpu.SemaphoreType.DMA((N_DEV,)),
            pltpu.SemaphoreType.DMA((N_DEV,)),
        ],
        compiler_params=pltpu.CompilerParams(collective_id=0),
    )(x, Wq, k2, v2, Wo)


# device time: 129862 ns/iter; 1.2850x vs baseline; 1.2850x over previous
import jax
import jax.numpy as jnp
from jax import lax
from jax.experimental import pallas as pl
from jax.experimental.pallas import tpu as pltpu

N_DEV = 8
B_LOC = 2
SQ = 256
SKV = 256
H_LOC = 4
DH = 64
DM = 512
HD = H_LOC * DH
BLK = 64


def kernel(x, Wq, K_ext, V_ext, Wo):
    k2 = K_ext.reshape(16, SKV, 32 * DH)
    v2 = V_ext.reshape(16, SKV, 32 * DH)
    wq_bf = Wq.astype(jnp.bfloat16)
    wo_bf = Wo.astype(jnp.bfloat16)

    def body(x_ref, wq_ref, k_hbm, v_hbm, wo_ref, out_ref,
             wq_bufs, wo_bufs, k_chunk, v_chunk,
             ksem, vsem, ssem_q, rsem_q, ssem_o, rsem_o):
        my = lax.axis_index("i")
        right = lax.rem(my + 1, N_DEV)
        left = lax.rem(my + N_DEV - 1, N_DEV)

        def start_kv_fetch(r):
            origin = lax.rem(my + N_DEV - r, N_DEV)
            col = origin * HD
            p = r % 2
            kcp = pltpu.make_async_copy(
                k_hbm.at[pl.ds(my * B_LOC, B_LOC), :, pl.ds(col, HD)],
                k_chunk.at[p], ksem.at[p])
            vcp = pltpu.make_async_copy(
                v_hbm.at[pl.ds(my * B_LOC, B_LOC), :, pl.ds(col, HD)],
                v_chunk.at[p], vsem.at[p])
            kcp.start()
            vcp.start()

        def wait_kv(r):
            p = r % 2
            pltpu.make_async_copy(
                k_hbm.at[pl.ds(0, B_LOC), :, pl.ds(0, HD)],
                k_chunk.at[p], ksem.at[p]).wait()
            pltpu.make_async_copy(
                v_hbm.at[pl.ds(0, B_LOC), :, pl.ds(0, HD)],
                v_chunk.at[p], vsem.at[p]).wait()

        start_kv_fetch(0)

        barrier_sem = pltpu.get_barrier_semaphore()
        for nbr in (left, right):
            pl.semaphore_signal(barrier_sem, inc=1, device_id=(nbr,),
                                device_id_type=pl.DeviceIdType.MESH)
        pl.semaphore_wait(barrier_sem, 2)

        wq_bufs[0] = wq_ref[...]
        wo_bufs[0] = wo_ref[...]

        qb = lax.broadcasted_iota(jnp.int32, (SQ, SKV), 0) // BLK
        kb = lax.broadcasted_iota(jnp.int32, (SQ, SKV), 1) // BLK
        mask = (qb == kb) | (kb == 0) | ((qb + kb) % 3 == 0)

        x2 = x_ref[...].reshape(B_LOC * SQ, DM).astype(jnp.bfloat16)

        def compute_chunk(r, acc):
            p = r % 2
            wq_c = wq_bufs[r]
            wo_c = wo_bufs[r]
            q_all = jnp.dot(x2, wq_c, preferred_element_type=jnp.float32)
            ctx_rows = []
            for b in range(B_LOC):
                parts = []
                for hh in range(H_LOC):
                    q_h = q_all[b * SQ:(b + 1) * SQ, hh * DH:(hh + 1) * DH]
                    k_h = k_chunk[p, b, :, hh * DH:(hh + 1) * DH]
                    v_h = v_chunk[p, b, :, hh * DH:(hh + 1) * DH]
                    s = lax.dot_general(
                        q_h, k_h, (((1,), (1,)), ((), ())),
                        preferred_element_type=jnp.float32) * 0.125
                    s = jnp.where(mask, s, -1e9)
                    m = jnp.max(s, axis=-1, keepdims=True)
                    w = jnp.exp(s - m)
                    w = w / jnp.sum(w, axis=-1, keepdims=True)
                    parts.append(jnp.dot(w, v_h,
                                         preferred_element_type=jnp.float32))
                ctx_rows.append(jnp.concatenate(parts, axis=1))
            ctx2 = jnp.concatenate(ctx_rows, axis=0)
            return acc + jnp.dot(ctx2.astype(jnp.bfloat16), wo_c,
                                 preferred_element_type=jnp.float32)

        acc = jnp.zeros((B_LOC * SQ, DM), jnp.float32)

        for h in range(N_DEV - 1):
            rq = pltpu.make_async_remote_copy(
                src_ref=wq_bufs.at[h], dst_ref=wq_bufs.at[h + 1],
                send_sem=ssem_q.at[h], recv_sem=rsem_q.at[h + 1],
                device_id=(right,), device_id_type=pl.DeviceIdType.MESH)
            ro = pltpu.make_async_remote_copy(
                src_ref=wo_bufs.at[h], dst_ref=wo_bufs.at[h + 1],
                send_sem=ssem_o.at[h], recv_sem=rsem_o.at[h + 1],
                device_id=(right,), device_id_type=pl.DeviceIdType.MESH)
            rq.start()
            ro.start()
            start_kv_fetch(h + 1)
            wait_kv(h)
            acc = compute_chunk(h, acc)
            rq.wait()
            ro.wait()

        wait_kv(N_DEV - 1)
        acc = compute_chunk(N_DEV - 1, acc)
        out_ref[...] = acc.reshape(B_LOC, SQ, DM)

    return pl.pallas_call(
        body,
        out_shape=jax.ShapeDtypeStruct((B_LOC, SQ, DM), jnp.float32),
        in_specs=[
            pl.BlockSpec(memory_space=pltpu.VMEM),
            pl.BlockSpec(memory_space=pltpu.VMEM),
            pl.BlockSpec(memory_space=pl.ANY),
            pl.BlockSpec(memory_space=pl.ANY),
            pl.BlockSpec(memory_space=pltpu.VMEM),
        ],
        out_specs=pl.BlockSpec(memory_space=pltpu.VMEM),
        scratch_shapes=[
            pltpu.VMEM((N_DEV, DM, HD), jnp.bfloat16),
            pltpu.VMEM((N_DEV, HD, DM), jnp.bfloat16),
            pltpu.VMEM((2, B_LOC, SKV, HD), jnp.float32),
            pltpu.VMEM((2, B_LOC, SKV, HD), jnp.float32),
            pltpu.SemaphoreType.DMA((2,)),
            pltpu.SemaphoreType.DMA((2,)),
            pltpu.SemaphoreType.DMA((N_DEV,)),
            pltpu.SemaphoreType.DMA((N_DEV,)),
            pltpu.SemaphoreType.DMA((N_DEV,)),
            pltpu.SemaphoreType.DMA((N_DEV,)),
        ],
        compiler_params=pltpu.CompilerParams(collective_id=0),
    )(x, wq_bf, k2, v2, wo_bf)


# device time: 129635 ns/iter; 1.2873x vs baseline; 1.0018x over previous
import jax
import jax.numpy as jnp
from jax import lax
from jax.experimental import pallas as pl
from jax.experimental.pallas import tpu as pltpu

N_DEV = 8
B_LOC = 2
SQ = 256
SKV = 256
H_LOC = 4
DH = 64
DM = 512
HD = H_LOC * DH
BLK = 64


def kernel(x, Wq, K_ext, V_ext, Wo):
    k2 = K_ext.reshape(16, SKV, 32 * DH)
    v2 = V_ext.reshape(16, SKV, 32 * DH)
    wq_bf = Wq.astype(jnp.bfloat16)
    wo_bf = Wo.astype(jnp.bfloat16)

    def body(x_ref, wq_ref, k_hbm, v_hbm, wo_ref, out_ref,
             wq_bufs, wo_bufs, k_chunk, v_chunk,
             ksem, vsem, ssem_q, rsem_q, ssem_o, rsem_o):
        my = lax.axis_index("i")
        right = lax.rem(my + 1, N_DEV)
        left = lax.rem(my + N_DEV - 1, N_DEV)

        def start_kv_fetch(r):
            origin = lax.rem(my + N_DEV - r, N_DEV)
            col = origin * HD
            p = r % 2
            kcp = pltpu.make_async_copy(
                k_hbm.at[pl.ds(my * B_LOC, B_LOC), :, pl.ds(col, HD)],
                k_chunk.at[p], ksem.at[p])
            vcp = pltpu.make_async_copy(
                v_hbm.at[pl.ds(my * B_LOC, B_LOC), :, pl.ds(col, HD)],
                v_chunk.at[p], vsem.at[p])
            kcp.start()
            vcp.start()

        def wait_kv(r):
            p = r % 2
            pltpu.make_async_copy(
                k_hbm.at[pl.ds(0, B_LOC), :, pl.ds(0, HD)],
                k_chunk.at[p], ksem.at[p]).wait()
            pltpu.make_async_copy(
                v_hbm.at[pl.ds(0, B_LOC), :, pl.ds(0, HD)],
                v_chunk.at[p], vsem.at[p]).wait()

        start_kv_fetch(0)

        barrier_sem = pltpu.get_barrier_semaphore()
        for nbr in (left, right):
            pl.semaphore_signal(barrier_sem, inc=1, device_id=(nbr,),
                                device_id_type=pl.DeviceIdType.MESH)
        pl.semaphore_wait(barrier_sem, 2)

        wq_bufs[0] = wq_ref[...]
        wo_bufs[0] = wo_ref[...]

        qb = lax.broadcasted_iota(jnp.int32, (SQ, SKV), 0) // BLK
        kb = lax.broadcasted_iota(jnp.int32, (SQ, SKV), 1) // BLK
        mask = (qb == kb) | (kb == 0) | ((qb + kb) % 3 == 0)

        x2 = x_ref[...].reshape(B_LOC * SQ, DM).astype(jnp.bfloat16)

        def compute_chunk(r, acc):
            p = r % 2
            wq_c = wq_bufs[r]
            wo_c = wo_bufs[r]
            q_all = jnp.dot(x2, wq_c, preferred_element_type=jnp.float32)
            ctx_rows = []
            for b in range(B_LOC):
                parts = []
                for hh in range(H_LOC):
                    q_h = q_all[b * SQ:(b + 1) * SQ,
                                hh * DH:(hh + 1) * DH].astype(jnp.bfloat16)
                    k_h = k_chunk[p, b, :,
                                  hh * DH:(hh + 1) * DH].astype(jnp.bfloat16)
                    v_h = v_chunk[p, b, :,
                                  hh * DH:(hh + 1) * DH].astype(jnp.bfloat16)
                    s = lax.dot_general(
                        q_h, k_h, (((1,), (1,)), ((), ())),
                        preferred_element_type=jnp.float32) * 0.125
                    s = jnp.where(mask, s, -1e9)
                    m = jnp.max(s, axis=-1, keepdims=True)
                    w = jnp.exp(s - m)
                    w = (w / jnp.sum(w, axis=-1, keepdims=True)
                         ).astype(jnp.bfloat16)
                    parts.append(jnp.dot(w, v_h,
                                         preferred_element_type=jnp.float32))
                ctx_rows.append(jnp.concatenate(parts, axis=1))
            ctx2 = jnp.concatenate(ctx_rows, axis=0)
            return acc + jnp.dot(ctx2.astype(jnp.bfloat16), wo_c,
                                 preferred_element_type=jnp.float32)

        acc = jnp.zeros((B_LOC * SQ, DM), jnp.float32)

        for h in range(N_DEV - 1):
            rq = pltpu.make_async_remote_copy(
                src_ref=wq_bufs.at[h], dst_ref=wq_bufs.at[h + 1],
                send_sem=ssem_q.at[h], recv_sem=rsem_q.at[h + 1],
                device_id=(right,), device_id_type=pl.DeviceIdType.MESH)
            ro = pltpu.make_async_remote_copy(
                src_ref=wo_bufs.at[h], dst_ref=wo_bufs.at[h + 1],
                send_sem=ssem_o.at[h], recv_sem=rsem_o.at[h + 1],
                device_id=(right,), device_id_type=pl.DeviceIdType.MESH)
            rq.start()
            ro.start()
            start_kv_fetch(h + 1)
            wait_kv(h)
            acc = compute_chunk(h, acc)
            rq.wait()
            ro.wait()

        wait_kv(N_DEV - 1)
        acc = compute_chunk(N_DEV - 1, acc)
        out_ref[...] = acc.reshape(B_LOC, SQ, DM)

    return pl.pallas_call(
        body,
        out_shape=jax.ShapeDtypeStruct((B_LOC, SQ, DM), jnp.float32),
        in_specs=[
            pl.BlockSpec(memory_space=pltpu.VMEM),
            pl.BlockSpec(memory_space=pltpu.VMEM),
            pl.BlockSpec(memory_space=pl.ANY),
            pl.BlockSpec(memory_space=pl.ANY),
            pl.BlockSpec(memory_space=pltpu.VMEM),
        ],
        out_specs=pl.BlockSpec(memory_space=pltpu.VMEM),
        scratch_shapes=[
            pltpu.VMEM((N_DEV, DM, HD), jnp.bfloat16),
            pltpu.VMEM((N_DEV, HD, DM), jnp.bfloat16),
            pltpu.VMEM((2, B_LOC, SKV, HD), jnp.float32),
            pltpu.VMEM((2, B_LOC, SKV, HD), jnp.float32),
            pltpu.SemaphoreType.DMA((2,)),
            pltpu.SemaphoreType.DMA((2,)),
            pltpu.SemaphoreType.DMA((N_DEV,)),
            pltpu.SemaphoreType.DMA((N_DEV,)),
            pltpu.SemaphoreType.DMA((N_DEV,)),
            pltpu.SemaphoreType.DMA((N_DEV,)),
        ],
        compiler_params=pltpu.CompilerParams(collective_id=0),
    )(x, wq_bf, k2, v2, wo_bf)


# device time: 93066 ns/iter; 1.7931x vs baseline; 1.3929x over previous
import os

import jax
import jax.numpy as jnp
from jax import lax
from jax.experimental import pallas as pl
from jax.experimental.pallas import tpu as pltpu

_MODE = os.environ.get("SCBAND_MODE", "full")

N_DEV = 8
B_LOC = 2
SQ = 256
SKV = 256
H_LOC = 4
DH = 64
DM = 512
HD = H_LOC * DH
BLK = 64


def kernel(x, Wq, K_ext, V_ext, Wo):
    k2 = K_ext.reshape(16, SKV, 32 * DH)
    v2 = V_ext.reshape(16, SKV, 32 * DH)
    wq_bf = Wq.astype(jnp.bfloat16)
    wo_bf = Wo.astype(jnp.bfloat16)

    def body(x_ref, wq_ref, k_hbm, v_hbm, wo_ref, out_ref,
             wq_bufs, wo_bufs, k_chunk, v_chunk,
             ksem, vsem, ssem_q, rsem_q, ssem_o, rsem_o):
        my = lax.axis_index("i")
        right = lax.rem(my + 1, N_DEV)
        left = lax.rem(my + N_DEV - 1, N_DEV)

        def start_kv_fetch(r):
            origin = lax.rem(my + N_DEV - r, N_DEV)
            col = origin * HD
            p = r % 2
            kcp = pltpu.make_async_copy(
                k_hbm.at[pl.ds(my * B_LOC, B_LOC), :, pl.ds(col, HD)],
                k_chunk.at[p], ksem.at[p])
            vcp = pltpu.make_async_copy(
                v_hbm.at[pl.ds(my * B_LOC, B_LOC), :, pl.ds(col, HD)],
                v_chunk.at[p], vsem.at[p])
            kcp.start()
            vcp.start()

        def wait_kv(r):
            p = r % 2
            pltpu.make_async_copy(
                k_hbm.at[pl.ds(0, B_LOC), :, pl.ds(0, HD)],
                k_chunk.at[p], ksem.at[p]).wait()
            pltpu.make_async_copy(
                v_hbm.at[pl.ds(0, B_LOC), :, pl.ds(0, HD)],
                v_chunk.at[p], vsem.at[p]).wait()

        start_kv_fetch(0)

        barrier_sem = pltpu.get_barrier_semaphore()
        for nbr in (left, right):
            pl.semaphore_signal(barrier_sem, inc=1, device_id=(nbr,),
                                device_id_type=pl.DeviceIdType.MESH)
        pl.semaphore_wait(barrier_sem, 2)

        wq_bufs[0] = wq_ref[...]
        wo_bufs[0] = wo_ref[...]

        qb = lax.broadcasted_iota(jnp.int32, (SQ, SKV), 0) // BLK
        kb = lax.broadcasted_iota(jnp.int32, (SQ, SKV), 1) // BLK
        mask = (qb == kb) | (kb == 0) | ((qb + kb) % 3 == 0)

        x2 = x_ref[...].reshape(B_LOC * SQ, DM).astype(jnp.bfloat16)

        def compute_chunk(r, acc):
            p = r % 2
            wq_c = wq_bufs[r]
            wo_c = wo_bufs[r]
            q_all = jnp.dot(x2, wq_c, preferred_element_type=jnp.float32)
            ctx_rows = []
            for b in range(B_LOC):
                parts = []
                for hh in range(H_LOC):
                    q_h = q_all[b * SQ:(b + 1) * SQ,
                                hh * DH:(hh + 1) * DH].astype(jnp.bfloat16)
                    k_h = k_chunk[p, b, :,
                                  hh * DH:(hh + 1) * DH].astype(jnp.bfloat16)
                    v_h = v_chunk[p, b, :,
                                  hh * DH:(hh + 1) * DH].astype(jnp.bfloat16)
                    s = lax.dot_general(
                        q_h, k_h, (((1,), (1,)), ((), ())),
                        preferred_element_type=jnp.float32) * 0.125
                    s = jnp.where(mask, s, -1e9)
                    m = jnp.max(s, axis=-1, keepdims=True)
                    w = jnp.exp(s - m)
                    w = (w / jnp.sum(w, axis=-1, keepdims=True)
                         ).astype(jnp.bfloat16)
                    parts.append(jnp.dot(w, v_h,
                                         preferred_element_type=jnp.float32))
                ctx_rows.append(jnp.concatenate(parts, axis=1))
            ctx2 = jnp.concatenate(ctx_rows, axis=0)
            return acc + jnp.dot(ctx2.astype(jnp.bfloat16), wo_c,
                                 preferred_element_type=jnp.float32)

        acc = jnp.zeros((B_LOC * SQ, DM), jnp.float32)

        for h in range(N_DEV - 1):
            if _MODE != "compute":
                rq = pltpu.make_async_remote_copy(
                    src_ref=wq_bufs.at[h], dst_ref=wq_bufs.at[h + 1],
                    send_sem=ssem_q.at[h], recv_sem=rsem_q.at[h + 1],
                    device_id=(right,), device_id_type=pl.DeviceIdType.MESH)
                ro = pltpu.make_async_remote_copy(
                    src_ref=wo_bufs.at[h], dst_ref=wo_bufs.at[h + 1],
                    send_sem=ssem_o.at[h], recv_sem=rsem_o.at[h + 1],
                    device_id=(right,), device_id_type=pl.DeviceIdType.MESH)
                rq.start()
                ro.start()
            start_kv_fetch(h + 1)
            wait_kv(h)
            if _MODE != "comm":
                acc = compute_chunk(h if _MODE == "full" else 0, acc)
            if _MODE != "compute":
                rq.wait()
                ro.wait()

        wait_kv(N_DEV - 1)
        if _MODE != "comm":
            acc = compute_chunk((N_DEV - 1) if _MODE == "full" else 0, acc)
        else:
            acc = acc + (wq_bufs[N_DEV - 1, 0, 0]
                         + wo_bufs[N_DEV - 1, 0, 0]).astype(jnp.float32)
        out_ref[...] = acc.reshape(B_LOC, SQ, DM)

    return pl.pallas_call(
        body,
        out_shape=jax.ShapeDtypeStruct((B_LOC, SQ, DM), jnp.float32),
        in_specs=[
            pl.BlockSpec(memory_space=pltpu.VMEM),
            pl.BlockSpec(memory_space=pltpu.VMEM),
            pl.BlockSpec(memory_space=pl.ANY),
            pl.BlockSpec(memory_space=pl.ANY),
            pl.BlockSpec(memory_space=pltpu.VMEM),
        ],
        out_specs=pl.BlockSpec(memory_space=pltpu.VMEM),
        scratch_shapes=[
            pltpu.VMEM((N_DEV, DM, HD), jnp.bfloat16),
            pltpu.VMEM((N_DEV, HD, DM), jnp.bfloat16),
            pltpu.VMEM((2, B_LOC, SKV, HD), jnp.float32),
            pltpu.VMEM((2, B_LOC, SKV, HD), jnp.float32),
            pltpu.SemaphoreType.DMA((2,)),
            pltpu.SemaphoreType.DMA((2,)),
            pltpu.SemaphoreType.DMA((N_DEV,)),
            pltpu.SemaphoreType.DMA((N_DEV,)),
            pltpu.SemaphoreType.DMA((N_DEV,)),
            pltpu.SemaphoreType.DMA((N_DEV,)),
        ],
        compiler_params=pltpu.CompilerParams(collective_id=0),
    )(x, wq_bf, k2, v2, wo_bf)


# device time: 87391 ns/iter; 1.9095x vs baseline; 1.0649x over previous
import os

import jax
import jax.numpy as jnp
from jax import lax
from jax.experimental import pallas as pl
from jax.experimental.pallas import tpu as pltpu

_MODE = os.environ.get("SCBAND_MODE", "full")

N_DEV = 8
B_LOC = 2
SQ = 256
SKV = 256
H_LOC = 4
DH = 64
DM = 512
HD = H_LOC * DH
BLK = 64


def kernel(x, Wq, K_ext, V_ext, Wo):
    k2 = K_ext.reshape(16, SKV, 32 * DH)
    v2 = V_ext.reshape(16, SKV, 32 * DH)
    wq_bf = Wq.astype(jnp.bfloat16)
    wo_bf = Wo.astype(jnp.bfloat16)

    def body(x_ref, wq_ref, k_hbm, v_hbm, wo_ref, out_ref,
             wq_bufs, wo_bufs, k_chunk, v_chunk,
             ksem, vsem, ssem_q, rsem_q, ssem_o, rsem_o):
        my = lax.axis_index("i")
        right = lax.rem(my + 1, N_DEV)
        left = lax.rem(my + N_DEV - 1, N_DEV)

        def start_kv_fetch(r):
            origin = lax.rem(my + N_DEV - r, N_DEV)
            col = origin * HD
            p = r % 2
            kcp = pltpu.make_async_copy(
                k_hbm.at[pl.ds(my * B_LOC, B_LOC), :, pl.ds(col, HD)],
                k_chunk.at[p], ksem.at[p])
            vcp = pltpu.make_async_copy(
                v_hbm.at[pl.ds(my * B_LOC, B_LOC), :, pl.ds(col, HD)],
                v_chunk.at[p], vsem.at[p])
            kcp.start()
            vcp.start()

        def wait_kv(r):
            p = r % 2
            pltpu.make_async_copy(
                k_hbm.at[pl.ds(0, B_LOC), :, pl.ds(0, HD)],
                k_chunk.at[p], ksem.at[p]).wait()
            pltpu.make_async_copy(
                v_hbm.at[pl.ds(0, B_LOC), :, pl.ds(0, HD)],
                v_chunk.at[p], vsem.at[p]).wait()

        if _MODE not in ("dots", "empty"):
            start_kv_fetch(0)

        barrier_sem = pltpu.get_barrier_semaphore()
        for nbr in (left, right):
            pl.semaphore_signal(barrier_sem, inc=1, device_id=(nbr,),
                                device_id_type=pl.DeviceIdType.MESH)
        pl.semaphore_wait(barrier_sem, 2)

        wq_bufs[0] = wq_ref[...]
        wo_bufs[0] = wo_ref[...]

        qb = lax.broadcasted_iota(jnp.int32, (SQ, SKV), 0) // BLK
        kb = lax.broadcasted_iota(jnp.int32, (SQ, SKV), 1) // BLK
        mask = (qb == kb) | (kb == 0) | ((qb + kb) % 3 == 0)

        x2 = x_ref[...].reshape(B_LOC * SQ, DM).astype(jnp.bfloat16)

        def compute_chunk(r, acc):
            p = r % 2
            wq_c = wq_bufs[r]
            wo_c = wo_bufs[r]
            q_all = jnp.dot(x2, wq_c, preferred_element_type=jnp.float32)
            if _MODE in ("inner0", "dots"):
                return acc + jnp.dot(q_all.astype(jnp.bfloat16), wo_c,
                                     preferred_element_type=jnp.float32)
            ctx_rows = []
            for b in range(B_LOC):
                parts = []
                for hh in range(H_LOC):
                    q_h = q_all[b * SQ:(b + 1) * SQ,
                                hh * DH:(hh + 1) * DH].astype(jnp.bfloat16)
                    k_h = k_chunk[p, b, :,
                                  hh * DH:(hh + 1) * DH].astype(jnp.bfloat16)
                    v_h = v_chunk[p, b, :,
                                  hh * DH:(hh + 1) * DH].astype(jnp.bfloat16)
                    s = lax.dot_general(
                        q_h, k_h, (((1,), (1,)), ((), ())),
                        preferred_element_type=jnp.float32) * 0.125
                    if _MODE == "nosm":
                        w = s.astype(jnp.bfloat16)
                    else:
                        s = jnp.where(mask, s, -1e9)
                        m = jnp.max(s, axis=-1, keepdims=True)
                        w = jnp.exp(s - m)
                        w = (w / jnp.sum(w, axis=-1, keepdims=True)
                             ).astype(jnp.bfloat16)
                    parts.append(jnp.dot(w, v_h,
                                         preferred_element_type=jnp.float32))
                ctx_rows.append(jnp.concatenate(parts, axis=1))
            ctx2 = jnp.concatenate(ctx_rows, axis=0)
            return acc + jnp.dot(ctx2.astype(jnp.bfloat16), wo_c,
                                 preferred_element_type=jnp.float32)

        acc = jnp.zeros((B_LOC * SQ, DM), jnp.float32)

        if _MODE in ("dots", "empty"):
            if _MODE == "dots":
                for h in range(N_DEV):
                    acc = compute_chunk(0, acc)
            out_ref[...] = acc.reshape(B_LOC, SQ, DM)
            return

        for h in range(N_DEV - 1):
            if _MODE not in ("compute", "nosm", "inner0"):
                rq = pltpu.make_async_remote_copy(
                    src_ref=wq_bufs.at[h], dst_ref=wq_bufs.at[h + 1],
                    send_sem=ssem_q.at[h], recv_sem=rsem_q.at[h + 1],
                    device_id=(right,), device_id_type=pl.DeviceIdType.MESH)
                ro = pltpu.make_async_remote_copy(
                    src_ref=wo_bufs.at[h], dst_ref=wo_bufs.at[h + 1],
                    send_sem=ssem_o.at[h], recv_sem=rsem_o.at[h + 1],
                    device_id=(right,), device_id_type=pl.DeviceIdType.MESH)
                rq.start()
                ro.start()
            start_kv_fetch(h + 1)
            wait_kv(h)
            if _MODE != "comm":
                acc = compute_chunk(h if _MODE == "full" else 0, acc)
            if _MODE not in ("compute", "nosm", "inner0"):
                rq.wait()
                ro.wait()

        wait_kv(N_DEV - 1)
        if _MODE != "comm":
            acc = compute_chunk((N_DEV - 1) if _MODE == "full" else 0, acc)
        else:
            acc = acc + (wq_bufs[N_DEV - 1, 0, 0]
                         + wo_bufs[N_DEV - 1, 0, 0]).astype(jnp.float32)
        out_ref[...] = acc.reshape(B_LOC, SQ, DM)

    return pl.pallas_call(
        body,
        out_shape=jax.ShapeDtypeStruct((B_LOC, SQ, DM), jnp.float32),
        in_specs=[
            pl.BlockSpec(memory_space=pltpu.VMEM),
            pl.BlockSpec(memory_space=pltpu.VMEM),
            pl.BlockSpec(memory_space=pl.ANY),
            pl.BlockSpec(memory_space=pl.ANY),
            pl.BlockSpec(memory_space=pltpu.VMEM),
        ],
        out_specs=pl.BlockSpec(memory_space=pltpu.VMEM),
        scratch_shapes=[
            pltpu.VMEM((N_DEV, DM, HD), jnp.bfloat16),
            pltpu.VMEM((N_DEV, HD, DM), jnp.bfloat16),
            pltpu.VMEM((2, B_LOC, SKV, HD), jnp.float32),
            pltpu.VMEM((2, B_LOC, SKV, HD), jnp.float32),
            pltpu.SemaphoreType.DMA((2,)),
            pltpu.SemaphoreType.DMA((2,)),
            pltpu.SemaphoreType.DMA((N_DEV,)),
            pltpu.SemaphoreType.DMA((N_DEV,)),
            pltpu.SemaphoreType.DMA((N_DEV,)),
            pltpu.SemaphoreType.DMA((N_DEV,)),
        ],
        compiler_params=pltpu.CompilerParams(collective_id=0),
    )(x, wq_bf, k2, v2, wo_bf)
